# baseline (speedup 1.0000x reference)
import jax
import jax.numpy as jnp
from jax.experimental import pallas as pl
from jax.experimental.pallas import tpu as pltpu

_NUM_SAMPLES = 100000
_NUM_FEATURES = 32
_BATCH = 1024
_INV_TEMP = 20.0

_BLOCK_N = 1024
_FULL = _NUM_SAMPLES // _BLOCK_N
_TAIL = _NUM_SAMPLES - _FULL * _BLOCK_N
_NBUF = 6


def _body(x_ref, f_hbm, o_hbm, obuf, fbuf, otail, ftail, *sems):
    load_sem = sems[:_NBUF]
    store_sem = sems[_NBUF:2 * _NBUF]
    tail_sem = sems[2 * _NBUF:]
    x = x_ref[...]
    nrm = jnp.sqrt(jnp.sum(x * x, axis=1, keepdims=True))
    xn = x * (_INV_TEMP / jnp.clip(nrm, 1e-12, None))

    def load(i):
        slot = i % _NBUF
        return pltpu.make_async_copy(
            f_hbm.at[pl.ds(i * _BLOCK_N, _BLOCK_N)],
            fbuf.at[slot],
            load_sem[slot])

    def store(i):
        slot = i % _NBUF
        return pltpu.make_async_copy(
            obuf.at[slot],
            o_hbm.at[:, pl.ds(i * _BLOCK_N, _BLOCK_N)],
            store_sem[slot])

    for k in range(_NBUF):
        load(k).start()

    for i in range(_FULL):
        slot = i % _NBUF
        load(i).wait()
        if i >= _NBUF:
            store(i - _NBUF).wait()
        obuf[slot] = jax.lax.dot_general(
            xn, fbuf[slot], (((1,), (1,)), ((), ())),
            preferred_element_type=jnp.float32)
        store(i).start()
        if i + _NBUF < _FULL:
            load(i + _NBUF).start()

    tail_load = pltpu.make_async_copy(
        f_hbm.at[pl.ds(_FULL * _BLOCK_N, _TAIL)], ftail, tail_sem[0])
    tail_load.start()
    tail_load.wait()
    otail[...] = jax.lax.dot_general(
        xn, ftail[...], (((1,), (1,)), ((), ())),
        preferred_element_type=jnp.float32)
    tail_store = pltpu.make_async_copy(
        otail, o_hbm.at[:, pl.ds(_FULL * _BLOCK_N, _TAIL)], tail_sem[1])
    tail_store.start()

    for k in range(_NBUF):
        store(_FULL - _NBUF + k).wait()
    tail_store.wait()


def kernel(inputs, targets, features):
    del targets
    return pl.pallas_call(
        _body,
        in_specs=[
            pl.BlockSpec((_BATCH, _NUM_FEATURES), lambda: (0, 0)),
            pl.BlockSpec(memory_space=pltpu.MemorySpace.HBM),
        ],
        out_specs=pl.BlockSpec(memory_space=pltpu.MemorySpace.HBM),
        out_shape=jax.ShapeDtypeStruct((_BATCH, _NUM_SAMPLES), jnp.float32),
        scratch_shapes=[
            pltpu.VMEM((_NBUF, _BATCH, _BLOCK_N), jnp.float32),
            pltpu.VMEM((_NBUF, _BLOCK_N, _NUM_FEATURES), jnp.float32),
            pltpu.VMEM((_BATCH, _TAIL), jnp.float32),
            pltpu.VMEM((_TAIL, _NUM_FEATURES), jnp.float32),
        ] + [pltpu.SemaphoreType.DMA] * (2 * _NBUF + 2),
    )(inputs, features)


# speedup vs baseline: 1.1221x; 1.1221x over previous
import jax
import jax.numpy as jnp
from jax.experimental import pallas as pl
from jax.experimental.pallas import tpu as pltpu

_NUM_SAMPLES = 100000
_NUM_FEATURES = 32
_BATCH = 1024
_INV_TEMP = 20.0

_BAND = 32
_NBANDS = _BATCH // _BAND
_CHUNK = 25088


def _mm_kernel(x_ref, ft_ref, o_ref):
    x = x_ref[...]
    nrm = jnp.sqrt(jnp.sum(x * x, axis=1, keepdims=True))
    xn = x * (_INV_TEMP / jnp.clip(nrm, 1e-12, None))
    c = 0
    while c < _NUM_SAMPLES:
        w = min(_CHUNK, _NUM_SAMPLES - c)
        o_ref[:, c:c + w] = jax.lax.dot_general(
            xn, ft_ref[:, c:c + w], (((1,), (0,)), ((), ())),
            preferred_element_type=jnp.float32)
        c += _CHUNK


def kernel(inputs, targets, features):
    del targets
    features_t = jnp.swapaxes(features, 0, 1)
    return pl.pallas_call(
        _mm_kernel,
        grid=(_NBANDS,),
        in_specs=[
            pl.BlockSpec((_BAND, _NUM_FEATURES), lambda i: (i, 0)),
            pl.BlockSpec(memory_space=pltpu.MemorySpace.VMEM),
        ],
        out_specs=pl.BlockSpec((_BAND, _NUM_SAMPLES), lambda i: (i, 0)),
        out_shape=jax.ShapeDtypeStruct((_BATCH, _NUM_SAMPLES), jnp.float32),
        compiler_params=pltpu.CompilerParams(
            dimension_semantics=("arbitrary",)),
    )(inputs, features_t)
